# baseline (speedup 1.0000x reference)
import functools

import jax
import jax.numpy as jnp
from jax import lax
from jax.experimental import pallas as pl
from jax.experimental.pallas import tpu as pltpu
from jax.experimental.pallas import tpu_sc as plsc

NE = 1024
D = 64
B = 32
T = 1024
N = B * T

NC = 2
NS = 16
NW = NC * NS
CHUNK = N // NW


def _vq_body(x_ref, w_ref, q_ref, idx_ref, loss_ref, wt, w2, acc):
    s = pl.program_id(0)

    @pl.when(s == 0)
    def _init():
        w = w_ref[...]
        acc[0, 0] = 0.0
        wt[...] = w.T
        w2[...] = w * jnp.float32(-2.0)

    row = jax.lax.broadcasted_iota(jnp.int32, (NE, T), 0)

    f_cb = x_ref[0]
    wsq = jnp.sum(w_ref[...] * w_ref[...], axis=1, keepdims=True)
    fsq = jnp.sum(f_cb * f_cb, axis=0, keepdims=True)
    mm2 = jax.lax.dot_general(
        w2[...], f_cb, (((1,), (0,)), ((), ())),
        preferred_element_type=jnp.float32)
    dist = (fsq + wsq) + mm2
    m = jnp.min(dist, axis=0, keepdims=True)
    idx = jnp.min(jnp.where(dist == m, row, NE),
                  axis=0, keepdims=True)
    oh = jnp.where(row == idx, jnp.float32(1.0), jnp.float32(0.0))
    q = jax.lax.dot_general(
        wt[...], oh, (((1,), (0,)), ((), ())),
        preferred_element_type=jnp.float32)

    acc[0, 0] += jnp.sum(m)
    q_ref[0] = q
    idx_ref[0, 0] = idx[0]

    @pl.when(s == B - 1)
    def _fin():
        loss_ref[0, 0] = 0.25 * acc[0, 0] / (N * D)


_sc_mesh = plsc.VectorSubcoreMesh(core_axis_name="c", subcore_axis_name="s")


@functools.partial(
    pl.kernel, mesh=_sc_mesh,
    out_type=jax.ShapeDtypeStruct((NC, NE), jnp.float32),
    scratch_types=[
        pltpu.VMEM((CHUNK,), jnp.int32),
        pltpu.VMEM((CHUNK,), jnp.float32),
        pltpu.VMEM_SHARED((NE,), jnp.float32),
    ],
)
def _sc_hist(idx_hbm, ones_hbm, zeros_hbm, out_hbm, idx_v, ones_v, shared):
    c = lax.axis_index("c")
    sub = lax.axis_index("s")
    wid = sub * NC + c
    base = wid * CHUNK
    pltpu.sync_copy(idx_hbm.at[pl.ds(base, CHUNK)], idx_v)
    pltpu.sync_copy(ones_hbm, ones_v)

    @pl.when(sub == 0)
    def _zero():
        pltpu.sync_copy(zeros_hbm, shared)

    plsc.subcore_barrier()
    pltpu.sync_copy(ones_v, shared.at[idx_v], add=True)
    plsc.subcore_barrier()

    @pl.when(sub == 0)
    def _emit():
        pltpu.sync_copy(shared, out_hbm.at[c])


def _perp_body(c_ref, perp_ref):
    p = (c_ref[0] + c_ref[1]) * jnp.float32(1.0 / N)
    perp_ref[0, 0] = jnp.exp(-jnp.sum(p * jnp.log(p + 1e-10)))


def kernel(x, W):
    x4 = x.reshape(B, D, T)
    q4, idx3, loss = pl.pallas_call(
        _vq_body,
        grid=(B,),
        in_specs=[
            pl.BlockSpec((1, D, T), lambda s: (s, 0, 0)),
            pl.BlockSpec((NE, D), lambda s: (0, 0)),
        ],
        out_specs=(
            pl.BlockSpec((1, D, T), lambda s: (s, 0, 0)),
            pl.BlockSpec((1, 1, T), lambda s: (s, 0, 0)),
            pl.BlockSpec(memory_space=pltpu.SMEM),
        ),
        out_shape=(
            jax.ShapeDtypeStruct((B, D, T), jnp.float32),
            jax.ShapeDtypeStruct((B, 1, T), jnp.int32),
            jax.ShapeDtypeStruct((1, 1), jnp.float32),
        ),
        scratch_shapes=[
            pltpu.VMEM((D, NE), jnp.float32),
            pltpu.VMEM((NE, D), jnp.float32),
            pltpu.SMEM((1, 1), jnp.float32),
        ],
    )(x4, W)

    cnt2 = _sc_hist(idx3.reshape(N),
                    jnp.ones((CHUNK,), jnp.float32),
                    jnp.zeros((NE,), jnp.float32))

    perp = pl.pallas_call(
        _perp_body,
        in_specs=[pl.BlockSpec((NC, NE), lambda: (0, 0))],
        out_specs=pl.BlockSpec(memory_space=pltpu.SMEM),
        out_shape=jax.ShapeDtypeStruct((1, 1), jnp.float32),
    )(cnt2)

    quantized = q4.reshape(32, 64, 32, 32)
    indices = idx3.reshape(32, 32, 32)
    return quantized, loss[0, 0], indices, perp[0, 0]


# speedup vs baseline: 1.1300x; 1.1300x over previous
import jax
import jax.numpy as jnp
from jax.experimental import pallas as pl
from jax.experimental.pallas import tpu as pltpu

NE = 1024
D = 64
B = 32
T = 1024
N = B * T


def _vq_body(x_ref, w_ref, q_ref, idx_ref, loss_ref, perp_ref,
             wt, w2, counts, acc):
    s = pl.program_id(0)

    @pl.when(s == 0)
    def _init():
        w = w_ref[...]
        counts[...] = jnp.zeros_like(counts)
        acc[0, 0] = 0.0
        wt[...] = w.T
        w2[...] = w * jnp.float32(-2.0)

    row = jax.lax.broadcasted_iota(jnp.int32, (NE, T), 0)

    f_cb = x_ref[0]
    wsq = jnp.sum(w_ref[...] * w_ref[...], axis=1, keepdims=True)
    fsq = jnp.sum(f_cb * f_cb, axis=0, keepdims=True)
    mm2 = jax.lax.dot_general(
        w2[...], f_cb, (((1,), (0,)), ((), ())),
        preferred_element_type=jnp.float32)
    dist = (fsq + wsq) + mm2
    m = jnp.min(dist, axis=0, keepdims=True)
    idx = jnp.min(jnp.where(dist == m, row, NE),
                  axis=0, keepdims=True)
    oh = jnp.where(row == idx, jnp.float32(1.0), jnp.float32(0.0))
    q = jax.lax.dot_general(
        wt[...], oh, (((1,), (0,)), ((), ())),
        preferred_element_type=jnp.float32)

    acc[0, 0] += jnp.sum(m)
    counts[...] += jnp.sum(oh, axis=1, keepdims=True)
    q_ref[0] = q
    idx_ref[0, 0] = idx[0]

    @pl.when(s == B - 1)
    def _fin():
        loss_ref[0, 0] = 0.25 * acc[0, 0] / (N * D)
        p = counts[...] / N
        perp_ref[0, 0] = jnp.exp(-jnp.sum(p * jnp.log(p + 1e-10)))


def kernel(x, W):
    x4 = x.reshape(B, D, T)
    q4, idx3, loss, perp = pl.pallas_call(
        _vq_body,
        grid=(B,),
        in_specs=[
            pl.BlockSpec((1, D, T), lambda s: (s, 0, 0)),
            pl.BlockSpec((NE, D), lambda s: (0, 0)),
        ],
        out_specs=(
            pl.BlockSpec((1, D, T), lambda s: (s, 0, 0)),
            pl.BlockSpec((1, 1, T), lambda s: (s, 0, 0)),
            pl.BlockSpec(memory_space=pltpu.SMEM),
            pl.BlockSpec(memory_space=pltpu.SMEM),
        ),
        out_shape=(
            jax.ShapeDtypeStruct((B, D, T), jnp.float32),
            jax.ShapeDtypeStruct((B, 1, T), jnp.int32),
            jax.ShapeDtypeStruct((1, 1), jnp.float32),
            jax.ShapeDtypeStruct((1, 1), jnp.float32),
        ),
        scratch_shapes=[
            pltpu.VMEM((D, NE), jnp.float32),
            pltpu.VMEM((NE, D), jnp.float32),
            pltpu.VMEM((NE, 1), jnp.float32),
            pltpu.SMEM((1, 1), jnp.float32),
        ],
    )(x4, W)
    quantized = q4.reshape(32, 64, 32, 32)
    indices = idx3.reshape(32, 32, 32)
    return quantized, loss[0, 0], indices, perp[0, 0]


# speedup vs baseline: 1.2772x; 1.1303x over previous
import jax
import jax.numpy as jnp
from jax.experimental import pallas as pl
from jax.experimental.pallas import tpu as pltpu

NE = 1024
D = 64
B = 32
T = 1024
N = B * T


def _vq_body(x_ref, w_ref, q_ref, idx_ref, loss_ref, perp_ref,
             wt, w2, counts, acc):
    s = pl.program_id(0)

    @pl.when(s == 0)
    def _init():
        w = w_ref[...]
        counts[...] = jnp.zeros_like(counts)
        acc[0, 0] = 0.0
        wt[...] = w.T
        w2[...] = w * jnp.float32(-2.0)

    row = jax.lax.broadcasted_iota(jnp.int32, (NE, T), 0)

    f_cb = x_ref[0]
    wsq = jnp.sum(w_ref[...] * w_ref[...], axis=1, keepdims=True)
    fsq = jnp.sum(f_cb * f_cb, axis=0, keepdims=True)
    mm2 = jax.lax.dot_general(
        w2[...], f_cb, (((1,), (0,)), ((), ())),
        preferred_element_type=jnp.float32)
    dist = (fsq + wsq) + mm2
    m = jnp.min(dist, axis=0, keepdims=True)
    idx = jnp.argmin(dist, axis=0).astype(jnp.int32)[None, :]
    oh = jnp.where(row == idx, jnp.float32(1.0), jnp.float32(0.0))
    q = jax.lax.dot_general(
        wt[...], oh, (((1,), (0,)), ((), ())),
        preferred_element_type=jnp.float32)

    acc[0, 0] += jnp.sum(m)
    counts[...] += jnp.sum(oh, axis=1, keepdims=True)
    q_ref[0] = q
    idx_ref[0, 0] = idx[0]

    @pl.when(s == B - 1)
    def _fin():
        loss_ref[0, 0] = 0.25 * acc[0, 0] / (N * D)
        p = counts[...] / N
        perp_ref[0, 0] = jnp.exp(-jnp.sum(p * jnp.log(p + 1e-10)))


def kernel(x, W):
    x4 = x.reshape(B, D, T)
    q4, idx3, loss, perp = pl.pallas_call(
        _vq_body,
        grid=(B,),
        in_specs=[
            pl.BlockSpec((1, D, T), lambda s: (s, 0, 0)),
            pl.BlockSpec((NE, D), lambda s: (0, 0)),
        ],
        out_specs=(
            pl.BlockSpec((1, D, T), lambda s: (s, 0, 0)),
            pl.BlockSpec((1, 1, T), lambda s: (s, 0, 0)),
            pl.BlockSpec(memory_space=pltpu.SMEM),
            pl.BlockSpec(memory_space=pltpu.SMEM),
        ),
        out_shape=(
            jax.ShapeDtypeStruct((B, D, T), jnp.float32),
            jax.ShapeDtypeStruct((B, 1, T), jnp.int32),
            jax.ShapeDtypeStruct((1, 1), jnp.float32),
            jax.ShapeDtypeStruct((1, 1), jnp.float32),
        ),
        scratch_shapes=[
            pltpu.VMEM((D, NE), jnp.float32),
            pltpu.VMEM((NE, D), jnp.float32),
            pltpu.VMEM((NE, 1), jnp.float32),
            pltpu.SMEM((1, 1), jnp.float32),
        ],
    )(x4, W)
    quantized = q4.reshape(32, 64, 32, 32)
    indices = idx3.reshape(32, 32, 32)
    return quantized, loss[0, 0], indices, perp[0, 0]


# speedup vs baseline: 1.2797x; 1.0020x over previous
import jax
import jax.numpy as jnp
from jax.experimental import pallas as pl
from jax.experimental.pallas import tpu as pltpu

NE = 1024
D = 64
B = 32
T = 1024
N = B * T


def _vq_body(x_ref, w_ref, q_ref, idx_ref, loss_ref, perp_ref,
             wt, w2, counts, acc):
    s = pl.program_id(0)

    @pl.when(s == 0)
    def _init():
        w = w_ref[...]
        counts[...] = jnp.zeros_like(counts)
        acc[0, 0] = 0.0
        wt[...] = w.T
        w2[...] = w * jnp.float32(-2.0)

    row = jax.lax.broadcasted_iota(jnp.int32, (NE, T), 0)

    f_cb = x_ref[0]
    wsq = jnp.sum(w_ref[...] * w_ref[...], axis=1, keepdims=True)
    fsq = jnp.sum(f_cb * f_cb, axis=0, keepdims=True)
    mm2 = jax.lax.dot_general(
        w2[...], f_cb, (((1,), (0,)), ((), ())),
        preferred_element_type=jnp.float32)
    dist = (fsq + wsq) + mm2
    idx = jnp.argmin(dist, axis=0).astype(jnp.int32)[None, :]
    oh = jnp.where(row == idx, jnp.float32(1.0), jnp.float32(0.0))
    q = jax.lax.dot_general(
        wt[...], oh, (((1,), (0,)), ((), ())),
        preferred_element_type=jnp.float32)

    err = q - f_cb
    acc[0, 0] += jnp.sum(err * err)
    counts[...] += jnp.sum(oh, axis=1, keepdims=True)
    q_ref[0] = q
    idx_ref[0, 0] = idx[0]

    @pl.when(s == B - 1)
    def _fin():
        loss_ref[0, 0] = 0.25 * acc[0, 0] / (N * D)
        p = counts[...] / N
        perp_ref[0, 0] = jnp.exp(-jnp.sum(p * jnp.log(p + 1e-10)))


def kernel(x, W):
    x4 = x.reshape(B, D, T)
    q4, idx3, loss, perp = pl.pallas_call(
        _vq_body,
        grid=(B,),
        in_specs=[
            pl.BlockSpec((1, D, T), lambda s: (s, 0, 0)),
            pl.BlockSpec((NE, D), lambda s: (0, 0)),
        ],
        out_specs=(
            pl.BlockSpec((1, D, T), lambda s: (s, 0, 0)),
            pl.BlockSpec((1, 1, T), lambda s: (s, 0, 0)),
            pl.BlockSpec(memory_space=pltpu.SMEM),
            pl.BlockSpec(memory_space=pltpu.SMEM),
        ),
        out_shape=(
            jax.ShapeDtypeStruct((B, D, T), jnp.float32),
            jax.ShapeDtypeStruct((B, 1, T), jnp.int32),
            jax.ShapeDtypeStruct((1, 1), jnp.float32),
            jax.ShapeDtypeStruct((1, 1), jnp.float32),
        ),
        scratch_shapes=[
            pltpu.VMEM((D, NE), jnp.float32),
            pltpu.VMEM((NE, D), jnp.float32),
            pltpu.VMEM((NE, 1), jnp.float32),
            pltpu.SMEM((1, 1), jnp.float32),
        ],
    )(x4, W)
    quantized = q4.reshape(32, 64, 32, 32)
    indices = idx3.reshape(32, 32, 32)
    return quantized, loss[0, 0], indices, perp[0, 0]
